# Initial kernel scaffold; baseline (speedup 1.0000x reference)
import jax
import jax.numpy as jnp
from jax.experimental import pallas as pl


def kernel(edge_attrs, params, edge_indices, triangles, inverse_edges):
    raise NotImplementedError("write your pallas kernel here")


# speedup vs baseline: 1.0408x; 1.0408x over previous
import functools

import jax
import jax.numpy as jnp
from jax.experimental import pallas as pl
from jax.experimental.pallas import tpu as pltpu

_N = 10000
_E = 160000
_T = 320000
_D = 128
_CHUNK = 1000

_INTERPRET = False


def _mm_kernel(x_ref, w_ref, b_ref, o_ref):
    o_ref[...] = (
        jnp.dot(x_ref[...], w_ref[...], preferred_element_type=jnp.float32)
        + b_ref[...]
    )


def _rowmm(x, W, b):
    n = x.shape[0]
    return pl.pallas_call(
        _mm_kernel,
        grid=(n // _CHUNK,),
        in_specs=[
            pl.BlockSpec((_CHUNK, _D), lambda i: (i, 0)),
            pl.BlockSpec((_D, _D), lambda i: (0, 0)),
            pl.BlockSpec((1, _D), lambda i: (0, 0)),
        ],
        out_specs=pl.BlockSpec((_CHUNK, _D), lambda i: (i, 0)),
        out_shape=jax.ShapeDtypeStruct((n, _D), jnp.float32),
        interpret=_INTERPRET,
    )(x, W, b.reshape(1, _D))


def _family_body(k, scale_ref, A_ref, *refs):
    term_refs = refs[:k]
    w_refs = refs[k : 2 * k]
    (bsum_ref, g_ref, beta_ref, w1_ref, b1_ref, w2_ref, b2_ref, o_ref) = refs[2 * k :]
    a = A_ref[...]
    acc = a * scale_ref[...] + bsum_ref[...]
    for t, w in zip(term_refs, w_refs):
        acc = acc + jnp.dot(t[...], w[...], preferred_element_type=jnp.float32)
    h = jnp.dot(acc, w1_ref[...], preferred_element_type=jnp.float32) + b1_ref[...]
    m = jnp.mean(h, axis=-1, keepdims=True)
    v = jnp.mean((h - m) ** 2, axis=-1, keepdims=True)
    h = (h - m) / jnp.sqrt(v + 1e-5) * g_ref[...] + beta_ref[...]
    h = jnp.maximum(h, 0.0)
    y = jnp.dot(h, w2_ref[...], preferred_element_type=jnp.float32) + b2_ref[...]
    o_ref[...] = jnp.maximum(y, 0.0) + a


def _family(A, terms, Ws, bias_sum, mlp, eps):
    n = A.shape[0]
    k = len(terms)
    chunk = _CHUNK
    row_spec = pl.BlockSpec((chunk, _D), lambda i: (i, 0))
    mat_spec = pl.BlockSpec((_D, _D), lambda i: (0, 0))
    vec_spec = pl.BlockSpec((1, _D), lambda i: (0, 0))
    scale = (1.0 + eps).reshape(1, 1)
    args = (
        [scale, A]
        + list(terms)
        + list(Ws)
        + [
            bias_sum.reshape(1, _D),
            mlp["gamma"].reshape(1, _D),
            mlp["beta"].reshape(1, _D),
            mlp["lin1"]["W"],
            mlp["lin1"]["b"].reshape(1, _D),
            mlp["lin2"]["W"],
            mlp["lin2"]["b"].reshape(1, _D),
        ]
    )
    in_specs = (
        [pl.BlockSpec((1, 1), lambda i: (0, 0)), row_spec]
        + [row_spec] * k
        + [mat_spec] * k
        + [vec_spec, vec_spec, vec_spec, mat_spec, vec_spec, mat_spec, vec_spec]
    )
    return pl.pallas_call(
        functools.partial(_family_body, k),
        grid=(n // chunk,),
        in_specs=in_specs,
        out_specs=row_spec,
        out_shape=jax.ShapeDtypeStruct((n, _D), jnp.float32),
        interpret=_INTERPRET,
    )(*args)


def kernel(edge_attrs, params, edge_indices, triangles, inverse_edges):
    A0, A1, A2 = edge_attrs
    p = params
    relu = jax.nn.relu
    seg = functools.partial(jax.ops.segment_sum)

    inner0 = _rowmm(A0, p["inner"]["W"], p["inner"]["b"])
    inner1 = _rowmm(A1, p["inner"]["W"], p["inner"]["b"])
    inner2 = _rowmm(A2, p["inner"]["W"], p["inner"]["b"])

    s1, e1 = edge_indices[0][0], edge_indices[0][1]
    s2, e2 = edge_indices[1][0], edge_indices[1][1]

    G1 = relu(inner0[s1] + inner0[e1])
    G2 = relu(inner0[s2] + inner0[e2])
    S10 = seg(relu(inner1), s1, num_segments=_N)
    S20 = seg(relu(inner2), s2, num_segments=_N)
    t = triangles[(1, 1, 1)]
    T111 = seg(relu(inner1[t[1]] + inner1[t[2]]), t[0], num_segments=_E)
    t = triangles[(2, 2, 2)]
    T222 = seg(relu(inner2[t[1]] + inner2[t[2]]), t[0], num_segments=_E)
    t = triangles[(1, 1, 2)]
    U = seg(relu(inner1[t[0]] + inner1[t[1]]), t[2], num_segments=_E)
    V = seg(relu(inner1[t[1]] + inner2[t[2]]), t[0], num_segments=_E)
    V2 = V + V[inverse_edges[0]]
    t = triangles[(2, 2, 1)]
    P = seg(relu(inner2[t[0]] + inner2[t[1]]), t[2], num_segments=_E)
    Q = seg(relu(inner2[t[1]] + inner1[t[2]]), t[0], num_segments=_E)
    Q2 = Q + Q[inverse_edges[1]]

    lins = p["lins"]
    w11, b11 = lins["(1, 1)"]["W"], lins["(1, 1)"]["b"]
    w10, b10 = lins["(1, 0)"]["W"], lins["(1, 0)"]["b"]
    w22, b22 = lins["(2, 2)"]["W"], lins["(2, 2)"]["b"]
    w20, b20 = lins["(2, 0)"]["W"], lins["(2, 0)"]["b"]
    w21, b21 = lins["(2, 1)"]["W"], lins["(2, 1)"]["b"]
    w12, b12 = lins["(1, 2)"]["W"], lins["(1, 2)"]["b"]
    eps = p["eps"]

    out0 = _family(
        A0, [S10, S20], [w10, w20], b10 + b20, p["mlps"]["0"], eps
    )
    out1 = _family(
        A1,
        [G1 + T111, P, V2],
        [w11, w22, w21],
        2.0 * b11 + b22 + b21,
        p["mlps"]["1"],
        eps,
    )
    out2 = _family(
        A2,
        [G2 + T222, U, Q2],
        [w22, w11, w12],
        2.0 * b22 + b11 + b12,
        p["mlps"]["2"],
        eps,
    )
    return (out0, out1, out2)
